# baseline (speedup 1.0000x reference)
import functools

import jax
import jax.numpy as jnp
import numpy as np
from jax import lax
from jax.experimental import pallas as pl
from jax.experimental.pallas import tpu as pltpu
from jax.experimental.pallas import tpu_sc as plsc

EMBED = 128
VOCAB = 21
NC, NS = 2, 16
NW = NC * NS
SLOT = 256
RG = 16


def _sc_lookup(w_flat, b, idx, per_w):
    groups = per_w // SLOT
    half = groups // 2
    mesh = plsc.VectorSubcoreMesh(core_axis_name="c", subcore_axis_name="s")

    @functools.partial(
        pl.kernel,
        out_type=jax.ShapeDtypeStruct((NW * groups, SLOT * EMBED), jnp.float32),
        mesh=mesh,
        compiler_params=pltpu.CompilerParams(needs_layout_passes=False),
        scratch_types=[
            pltpu.VMEM((per_w,), jnp.int32),
            pltpu.VMEM((VOCAB * EMBED,), jnp.float32),
            pltpu.VMEM((EMBED,), jnp.float32),
            pltpu.VMEM((VOCAB * EMBED,), jnp.float32),
            pltpu.VMEM((SLOT * EMBED,), jnp.float32),
            pltpu.VMEM((SLOT * EMBED,), jnp.float32),
            pltpu.SemaphoreType.DMA,
            pltpu.SemaphoreType.DMA,
        ],
    )
    def k(w_hbm, b_hbm, idx_hbm, out_hbm, idx_v, w_v, b_v, tab_v, rows0,
          rows1, s0, s1):
        rows = (rows0, rows1)
        ssem = (s0, s1)
        iota = lax.iota(jnp.int32, 16)
        wid = lax.axis_index("s") * NC + lax.axis_index("c")
        gbase = wid * groups
        pltpu.sync_copy(idx_hbm.at[wid], idx_v)
        pltpu.sync_copy(w_hbm, w_v)
        pltpu.sync_copy(b_hbm, b_v)

        bvecs = [b_v[pl.ds(e8 * 16, 16)] for e8 in range(8)]
        for v in range(VOCAB):
            for e8 in range(8):
                widx = (e8 * 16 + iota) * VOCAB + v
                col = plsc.load_gather(w_v, [widx])
                tab_v[pl.ds(v * EMBED + e8 * 16, 16)] = col + bvecs[e8]

        def expand(slot, g):
            def body(rb, carry):
                r0 = g * SLOT + rb * RG
                idxv = idx_v[pl.ds(r0, RG)]
                basev = idxv * EMBED
                scv = iota * EMBED + rb * (RG * EMBED)

                @plsc.parallel_loop(0, EMBED, unroll=EMBED)
                def _(e):
                    col = plsc.load_gather(tab_v, [basev + e])
                    plsc.store_scatter(rows[slot], [scv + e], col)

                return carry

            lax.fori_loop(0, SLOT // RG, body, 0)

        def scat(slot, g):
            pltpu.async_copy(rows[slot], out_hbm.at[gbase + g], ssem[slot])

        def scat_wait(slot, g):
            pltpu.make_async_copy(rows[slot], out_hbm.at[gbase + g],
                                  ssem[slot]).wait()

        def body(h, carry):
            ga = 2 * h
            gb = 2 * h + 1

            @pl.when(h >= 1)
            def _():
                scat_wait(0, ga - 2)

            expand(0, ga)
            scat(0, ga)

            @pl.when(h >= 1)
            def _():
                scat_wait(1, gb - 2)

            expand(1, gb)
            scat(1, gb)
            return carry

        lax.fori_loop(0, half, body, 0)
        scat_wait(0, 2 * half - 2)
        scat_wait(1, 2 * half - 1)

    return k(w_flat, b, idx)


def kernel(seq_ids, W, b):
    B, L = seq_ids.shape
    n = B * L
    per_w = n // NW
    idx = seq_ids.reshape(NW, per_w).astype(jnp.int32)
    out = _sc_lookup(W.reshape(-1), b, idx, per_w)
    return out.reshape(B, L, EMBED)


# speedup vs baseline: 2.7974x; 2.7974x over previous
import functools

import jax
import jax.numpy as jnp
import numpy as np
from jax import lax
from jax.experimental import pallas as pl
from jax.experimental.pallas import tpu as pltpu
from jax.experimental.pallas import tpu_sc as plsc

EMBED = 128
VOCAB = 21
NC, NS = 2, 16
NW = NC * NS
SLOT = 256
RG = 16


def _sc_lookup(w_flat, b, idx, per_w):
    groups = per_w // SLOT
    half = groups // 2
    mesh = plsc.VectorSubcoreMesh(core_axis_name="c", subcore_axis_name="s")

    @functools.partial(
        pl.kernel,
        out_type=jax.ShapeDtypeStruct((NW * groups, SLOT * EMBED), jnp.float32),
        mesh=mesh,
        compiler_params=pltpu.CompilerParams(needs_layout_passes=False),
        scratch_types=[
            pltpu.VMEM((per_w,), jnp.int32),
            pltpu.VMEM((VOCAB * EMBED,), jnp.float32),
            pltpu.VMEM((EMBED,), jnp.float32),
            pltpu.VMEM((VOCAB * EMBED,), jnp.float32),
            pltpu.VMEM((SLOT * EMBED,), jnp.float32),
            pltpu.VMEM((SLOT * EMBED,), jnp.float32),
            pltpu.SemaphoreType.DMA,
            pltpu.SemaphoreType.DMA,
        ],
    )
    def k(w_hbm, b_hbm, idx_hbm, out_hbm, idx_v, w_v, b_v, tab_v, rows0,
          rows1, s0, s1):
        rows = (rows0, rows1)
        ssem = (s0, s1)
        iota = lax.iota(jnp.int32, 16)
        wid = lax.axis_index("s") * NC + lax.axis_index("c")
        gbase = wid * groups
        pltpu.sync_copy(idx_hbm.at[wid], idx_v)
        pltpu.sync_copy(w_hbm, w_v)
        pltpu.sync_copy(b_hbm, b_v)

        bvecs = [b_v[pl.ds(e8 * 16, 16)] for e8 in range(8)]
        for v in range(VOCAB):
            for e8 in range(8):
                widx = (e8 * 16 + iota) * VOCAB + v
                col = plsc.load_gather(w_v, [widx])
                tab_v[pl.ds(v * EMBED + e8 * 16, 16)] = col + bvecs[e8]

        def expand(slot, g):
            @plsc.parallel_loop(0, SLOT // RG, unroll=4)
            def _(rb):
                r0 = g * SLOT + rb * RG
                idxv = idx_v[pl.ds(r0, RG)]
                for rr in range(RG):
                    base = idxv[rr] * EMBED + iota
                    dst0 = (rb * RG + rr) * EMBED
                    for e8 in range(8):
                        col = plsc.load_gather(tab_v, [base + e8 * 16])
                        rows[slot][pl.ds(dst0 + e8 * 16, 16)] = col

        def scat(slot, g):
            pltpu.async_copy(rows[slot], out_hbm.at[gbase + g], ssem[slot])

        def scat_wait(slot, g):
            pltpu.make_async_copy(rows[slot], out_hbm.at[gbase + g],
                                  ssem[slot]).wait()

        def body(h, carry):
            ga = 2 * h
            gb = 2 * h + 1

            @pl.when(h >= 1)
            def _():
                scat_wait(0, ga - 2)

            expand(0, ga)
            scat(0, ga)

            @pl.when(h >= 1)
            def _():
                scat_wait(1, gb - 2)

            expand(1, gb)
            scat(1, gb)
            return carry

        lax.fori_loop(0, half, body, 0)
        scat_wait(0, 2 * half - 2)
        scat_wait(1, 2 * half - 1)

    return k(w_flat, b, idx)


def kernel(seq_ids, W, b):
    B, L = seq_ids.shape
    n = B * L
    per_w = n // NW
    idx = seq_ids.reshape(NW, per_w).astype(jnp.int32)
    out = _sc_lookup(W.reshape(-1), b, idx, per_w)
    return out.reshape(B, L, EMBED)


# speedup vs baseline: 3.3822x; 1.2091x over previous
import functools

import jax
import jax.numpy as jnp
import numpy as np
from jax import lax
from jax.experimental import pallas as pl
from jax.experimental.pallas import tpu as pltpu
from jax.experimental.pallas import tpu_sc as plsc

EMBED = 128
VOCAB = 21
NC, NS = 2, 16
NW = NC * NS
SLOT = 256
RG = 16


def _sc_lookup(w_flat, b, idx, per_w):
    groups = per_w // SLOT
    half = groups // 2
    mesh = plsc.VectorSubcoreMesh(core_axis_name="c", subcore_axis_name="s")

    @functools.partial(
        pl.kernel,
        out_type=jax.ShapeDtypeStruct((NW * groups, SLOT * EMBED), jnp.float32),
        mesh=mesh,
        compiler_params=pltpu.CompilerParams(needs_layout_passes=False),
        scratch_types=[
            pltpu.VMEM((per_w,), jnp.int32),
            pltpu.VMEM((VOCAB * EMBED,), jnp.float32),
            pltpu.VMEM((EMBED,), jnp.float32),
            pltpu.VMEM((VOCAB * EMBED,), jnp.float32),
            pltpu.VMEM((SLOT * EMBED,), jnp.float32),
            pltpu.VMEM((SLOT * EMBED,), jnp.float32),
            pltpu.SemaphoreType.DMA,
            pltpu.SemaphoreType.DMA,
        ],
    )
    def k(w_hbm, b_hbm, idx_hbm, out_hbm, idx_v, w_v, b_v, tab_v, rows0,
          rows1, s0, s1):
        rows = (rows0, rows1)
        ssem = (s0, s1)
        iota = lax.iota(jnp.int32, 16)
        wid = lax.axis_index("s") * NC + lax.axis_index("c")
        gbase = wid * groups
        pltpu.sync_copy(idx_hbm.at[wid], idx_v)
        pltpu.sync_copy(w_hbm, w_v)
        pltpu.sync_copy(b_hbm, b_v)

        bvecs = [b_v[pl.ds(e8 * 16, 16)] for e8 in range(8)]
        for v in range(VOCAB):
            for e8 in range(8):
                widx = (e8 * 16 + iota) * VOCAB + v
                col = plsc.load_gather(w_v, [widx])
                tab_v[pl.ds(v * EMBED + e8 * 16, 16)] = col + bvecs[e8]

        def expand(slot, g):
            @plsc.parallel_loop(0, SLOT // RG, unroll=1)
            def _(rb):
                r0 = g * SLOT + rb * RG
                idxv = idx_v[pl.ds(r0, RG)]
                for rr in range(RG):
                    base = idxv[rr] * EMBED + iota
                    dst0 = (rb * RG + rr) * EMBED
                    for e8 in range(8):
                        col = plsc.load_gather(tab_v, [base + e8 * 16])
                        rows[slot][pl.ds(dst0 + e8 * 16, 16)] = col

        def scat(slot, g):
            pltpu.async_copy(rows[slot], out_hbm.at[gbase + g], ssem[slot])

        def scat_wait(slot, g):
            pltpu.make_async_copy(rows[slot], out_hbm.at[gbase + g],
                                  ssem[slot]).wait()

        def body(h, carry):
            ga = 2 * h
            gb = 2 * h + 1

            @pl.when(h >= 1)
            def _():
                scat_wait(0, ga - 2)

            expand(0, ga)
            scat(0, ga)

            @pl.when(h >= 1)
            def _():
                scat_wait(1, gb - 2)

            expand(1, gb)
            scat(1, gb)
            return carry

        lax.fori_loop(0, half, body, 0)
        scat_wait(0, 2 * half - 2)
        scat_wait(1, 2 * half - 1)

    return k(w_flat, b, idx)


def kernel(seq_ids, W, b):
    B, L = seq_ids.shape
    n = B * L
    per_w = n // NW
    idx = seq_ids.reshape(NW, per_w).astype(jnp.int32)
    out = _sc_lookup(W.reshape(-1), b, idx, per_w)
    return out.reshape(B, L, EMBED)


# speedup vs baseline: 3.7280x; 1.1022x over previous
import functools

import jax
import jax.numpy as jnp
import numpy as np
from jax import lax
from jax.experimental import pallas as pl
from jax.experimental.pallas import tpu as pltpu
from jax.experimental.pallas import tpu_sc as plsc

EMBED = 128
VOCAB = 21
NC, NS = 2, 16
NW = NC * NS
SLOT = 256
RG = 16


def _sc_lookup(w_flat, b, idx, per_w):
    groups = per_w // SLOT
    half = groups // 2
    mesh = plsc.VectorSubcoreMesh(core_axis_name="c", subcore_axis_name="s")

    @functools.partial(
        pl.kernel,
        out_type=jax.ShapeDtypeStruct((NW * groups, SLOT * EMBED), jnp.float32),
        mesh=mesh,
        compiler_params=pltpu.CompilerParams(needs_layout_passes=False),
        scratch_types=[
            pltpu.VMEM((per_w,), jnp.int32),
            pltpu.VMEM((VOCAB * EMBED,), jnp.float32),
            pltpu.VMEM((EMBED,), jnp.float32),
            pltpu.VMEM((VOCAB * EMBED,), jnp.float32),
            pltpu.VMEM((SLOT * EMBED,), jnp.float32),
            pltpu.VMEM((SLOT * EMBED,), jnp.float32),
            pltpu.SemaphoreType.DMA,
            pltpu.SemaphoreType.DMA,
        ],
    )
    def k(w_hbm, b_hbm, idx_hbm, out_hbm, idx_v, w_v, b_v, tab_v, rows0,
          rows1, s0, s1):
        rows = (rows0, rows1)
        ssem = (s0, s1)
        iota = lax.iota(jnp.int32, 16)
        wid = lax.axis_index("s") * NC + lax.axis_index("c")
        gbase = wid * groups
        pltpu.sync_copy(idx_hbm.at[wid], idx_v)
        pltpu.sync_copy(w_hbm, w_v)
        pltpu.sync_copy(b_hbm, b_v)

        bvecs = [b_v[pl.ds(e8 * 16, 16)] for e8 in range(8)]
        for v in range(VOCAB):
            for e8 in range(8):
                widx = (e8 * 16 + iota) * VOCAB + v
                col = plsc.load_gather(w_v, [widx])
                tab_v[pl.ds(v * EMBED + e8 * 16, 16)] = col + bvecs[e8]

        def expand(slot, g):
            @plsc.parallel_loop(0, SLOT // RG, unroll=1)
            def _(rb):
                r0 = g * SLOT + rb * RG
                idxv = idx_v[pl.ds(r0, RG)]
                for rr in range(RG):
                    sbase = idxv[rr] * EMBED
                    dst0 = (rb * RG + rr) * EMBED
                    for e8 in range(8):
                        col = tab_v[pl.ds(sbase + e8 * 16, 16)]
                        rows[slot][pl.ds(dst0 + e8 * 16, 16)] = col

        def scat(slot, g):
            pltpu.async_copy(rows[slot], out_hbm.at[gbase + g], ssem[slot])

        def scat_wait(slot, g):
            pltpu.make_async_copy(rows[slot], out_hbm.at[gbase + g],
                                  ssem[slot]).wait()

        def body(h, carry):
            ga = 2 * h
            gb = 2 * h + 1

            @pl.when(h >= 1)
            def _():
                scat_wait(0, ga - 2)

            expand(0, ga)
            scat(0, ga)

            @pl.when(h >= 1)
            def _():
                scat_wait(1, gb - 2)

            expand(1, gb)
            scat(1, gb)
            return carry

        lax.fori_loop(0, half, body, 0)
        scat_wait(0, 2 * half - 2)
        scat_wait(1, 2 * half - 1)

    return k(w_flat, b, idx)


def kernel(seq_ids, W, b):
    B, L = seq_ids.shape
    n = B * L
    per_w = n // NW
    idx = seq_ids.reshape(NW, per_w).astype(jnp.int32)
    out = _sc_lookup(W.reshape(-1), b, idx, per_w)
    return out.reshape(B, L, EMBED)


# speedup vs baseline: 10.2543x; 2.7506x over previous
import functools

import jax
import jax.numpy as jnp
from jax import lax
from jax.experimental import pallas as pl
from jax.experimental.pallas import tpu as pltpu
from jax.experimental.pallas import tpu_sc as plsc

EMBED = 128
VOCAB = 21
NC, NS = 2, 16
NW = NC * NS
CHUNK = 128
SLOT = 256


def _sc_lookup(w_flat, b, idx, per_w):
    n_chunks = per_w // CHUNK
    groups = per_w // SLOT
    half = groups // 2
    mesh = plsc.VectorSubcoreMesh(core_axis_name="c", subcore_axis_name="s")

    @functools.partial(
        pl.kernel,
        out_type=jax.ShapeDtypeStruct((NW * groups, SLOT, EMBED), jnp.float32),
        mesh=mesh,
        compiler_params=pltpu.CompilerParams(needs_layout_passes=False),
        scratch_types=[
            pltpu.VMEM((n_chunks, CHUNK), jnp.int32),
            pltpu.VMEM((VOCAB * EMBED,), jnp.float32),
            pltpu.VMEM((EMBED,), jnp.float32),
            pltpu.VMEM((VOCAB, EMBED), jnp.float32),
            pltpu.VMEM((SLOT, EMBED), jnp.float32),
            pltpu.VMEM((SLOT, EMBED), jnp.float32),
            pltpu.VMEM_SHARED((VOCAB, EMBED), jnp.float32),
            pltpu.SemaphoreType.DMA,
            pltpu.SemaphoreType.DMA,
            pltpu.SemaphoreType.DMA,
            pltpu.SemaphoreType.DMA,
        ],
    )
    def k(w_hbm, b_hbm, idx_hbm, out_hbm, idx_v, w_v, b_v, tab_v, rows0,
          rows1, shtab, g0, g1, s0, s1):
        rows = (rows0, rows1)
        gsem = (g0, g1)
        ssem = (s0, s1)
        iota = lax.iota(jnp.int32, 16)
        sid = lax.axis_index("s")
        wid = sid * NC + lax.axis_index("c")
        gbase = wid * groups
        pltpu.sync_copy(idx_hbm.at[wid], idx_v)
        pltpu.sync_copy(w_hbm, w_v)
        pltpu.sync_copy(b_hbm, b_v)

        bvecs = [b_v[pl.ds(e8 * 16, 16)] for e8 in range(8)]
        for v in range(VOCAB):
            for e8 in range(8):
                widx = (e8 * 16 + iota) * VOCAB + v
                col = plsc.load_gather(w_v, [widx])
                tab_v[v, pl.ds(e8 * 16, 16)] = col + bvecs[e8]

        @pl.when(sid == 0)
        def _():
            pltpu.sync_copy(tab_v, shtab)

        plsc.subcore_barrier()

        def fire(slot, g):
            for i in range(SLOT // CHUNK):
                pltpu.async_copy(shtab.at[idx_v.at[g * 2 + i]],
                                 rows[slot].at[pl.ds(i * CHUNK, CHUNK)],
                                 gsem[slot])

        def drain(slot, g):
            for i in range(SLOT // CHUNK):
                pltpu.make_async_copy(shtab.at[idx_v.at[g * 2 + i]],
                                      rows[slot].at[pl.ds(i * CHUNK, CHUNK)],
                                      gsem[slot]).wait()

        def scat(slot, g):
            pltpu.async_copy(rows[slot], out_hbm.at[gbase + g], ssem[slot])

        def scat_wait(slot, g):
            pltpu.make_async_copy(rows[slot], out_hbm.at[gbase + g],
                                  ssem[slot]).wait()

        def body(h, carry):
            ga = 2 * h
            gb = 2 * h + 1

            @pl.when(h >= 1)
            def _():
                scat_wait(0, ga - 2)

            fire(0, ga)

            @pl.when(h >= 1)
            def _():
                scat_wait(1, gb - 2)

            fire(1, gb)
            drain(0, ga)
            scat(0, ga)
            drain(1, gb)
            scat(1, gb)
            return carry

        lax.fori_loop(0, half, body, 0)
        scat_wait(0, 2 * half - 2)
        scat_wait(1, 2 * half - 1)

    return k(w_flat, b, idx)


def kernel(seq_ids, W, b):
    B, L = seq_ids.shape
    n = B * L
    per_w = n // NW
    idx = seq_ids.reshape(NW, per_w // CHUNK, CHUNK).astype(jnp.int32)
    out = _sc_lookup(W.reshape(-1), b, idx, per_w)
    return out.reshape(B, L, EMBED)


# speedup vs baseline: 14.7324x; 1.4367x over previous
import functools

import jax
import jax.numpy as jnp
from jax import lax
from jax.experimental import pallas as pl
from jax.experimental.pallas import tpu as pltpu
from jax.experimental.pallas import tpu_sc as plsc

EMBED = 128
VOCAB = 21
NC, NS = 2, 16
NW = NC * NS
CHUNK = 128
NBUF = 4


def _sc_lookup(w_flat, b, idx, per_w):
    n_chunks = per_w // CHUNK
    half = n_chunks // NBUF
    mesh = plsc.VectorSubcoreMesh(core_axis_name="c", subcore_axis_name="s")

    @functools.partial(
        pl.kernel,
        out_type=jax.ShapeDtypeStruct((NW * n_chunks, CHUNK, EMBED),
                                      jnp.float32),
        mesh=mesh,
        compiler_params=pltpu.CompilerParams(needs_layout_passes=False),
        scratch_types=[
            pltpu.VMEM((n_chunks, CHUNK), jnp.int32),
            pltpu.VMEM((VOCAB * EMBED,), jnp.float32),
            pltpu.VMEM((EMBED,), jnp.float32),
            pltpu.VMEM((VOCAB, EMBED), jnp.float32),
            pltpu.VMEM((CHUNK, EMBED), jnp.float32),
            pltpu.VMEM((CHUNK, EMBED), jnp.float32),
            pltpu.VMEM((CHUNK, EMBED), jnp.float32),
            pltpu.VMEM((CHUNK, EMBED), jnp.float32),
            pltpu.VMEM_SHARED((VOCAB, EMBED), jnp.float32),
            pltpu.SemaphoreType.DMA,
            pltpu.SemaphoreType.DMA,
            pltpu.SemaphoreType.DMA,
            pltpu.SemaphoreType.DMA,
            pltpu.SemaphoreType.DMA,
            pltpu.SemaphoreType.DMA,
            pltpu.SemaphoreType.DMA,
            pltpu.SemaphoreType.DMA,
        ],
    )
    def k(w_hbm, b_hbm, idx_hbm, out_hbm, idx_v, w_v, b_v, tab_v, rows0,
          rows1, rows2, rows3, shtab, g0, g1, g2, g3, s0, s1, s2, s3):
        rows = (rows0, rows1, rows2, rows3)
        gsem = (g0, g1, g2, g3)
        ssem = (s0, s1, s2, s3)
        iota = lax.iota(jnp.int32, 16)
        sid = lax.axis_index("s")
        wid = sid * NC + lax.axis_index("c")
        gbase = wid * n_chunks
        pltpu.sync_copy(idx_hbm.at[wid], idx_v)
        pltpu.sync_copy(w_hbm, w_v)
        pltpu.sync_copy(b_hbm, b_v)

        bvecs = [b_v[pl.ds(e8 * 16, 16)] for e8 in range(8)]
        for v in range(VOCAB):
            for e8 in range(8):
                widx = (e8 * 16 + iota) * VOCAB + v
                col = plsc.load_gather(w_v, [widx])
                tab_v[v, pl.ds(e8 * 16, 16)] = col + bvecs[e8]

        @pl.when(sid == 0)
        def _():
            pltpu.sync_copy(tab_v, shtab)

        plsc.subcore_barrier()

        def fire(slot, g):
            pltpu.async_copy(shtab.at[idx_v.at[g]], rows[slot], gsem[slot])

        def drain(slot, g):
            pltpu.make_async_copy(shtab.at[idx_v.at[g]], rows[slot],
                                  gsem[slot]).wait()

        def scat(slot, g):
            pltpu.async_copy(rows[slot], out_hbm.at[gbase + g], ssem[slot])

        def scat_wait(slot, g):
            pltpu.make_async_copy(rows[slot], out_hbm.at[gbase + g],
                                  ssem[slot]).wait()

        def body(h, carry):
            g0h = NBUF * h
            for i in range(NBUF):
                @pl.when(h >= 1)
                def _(i=i):
                    scat_wait(i, g0h + i - NBUF)

                fire(i, g0h + i)
            for i in range(NBUF):
                drain(i, g0h + i)
                scat(i, g0h + i)
            return carry

        lax.fori_loop(0, half, body, 0)
        for i in range(NBUF):
            scat_wait(i, NBUF * (half - 1) + i)

    return k(w_flat, b, idx)


def kernel(seq_ids, W, b):
    B, L = seq_ids.shape
    n = B * L
    per_w = n // NW
    idx = seq_ids.reshape(NW, per_w // CHUNK, CHUNK).astype(jnp.int32)
    out = _sc_lookup(W.reshape(-1), b, idx, per_w)
    return out.reshape(B, L, EMBED)
